# baseline (speedup 1.0000x reference)
import functools

import jax
import jax.numpy as jnp
from jax import lax
from jax.experimental import pallas as pl
from jax.experimental.pallas import tpu as pltpu
from jax.experimental.pallas import tpu_sc as plsc

_B, _C, _H, _W = 4, 96, 224, 224

_C_SC = 96

_CB = 16
_K = (_C - _C_SC) // _CB
_KOFF = _C_SC // _CB


def _mse_body(o_ref, m_ref, g_ref, out_ref):
    k = pl.program_id(1)

    @pl.when(k == 0)
    def _init():
        out_ref[...] = jnp.zeros_like(out_ref)

    o = o_ref[...]
    m = m_ref[...]
    g = g_ref[...]
    d = (o - g) * m
    s1v = jnp.sum(d * d, axis=(0, 1, 2))
    s2v = jnp.sum(m, axis=(0, 1, 2))
    out_ref[0, 0, :] += s1v
    out_ref[0, 1, :] += s2v


def _tc_partial_sums(o, m, g, interpret=False):
    spec = pl.BlockSpec((1, _CB, _H, _W), lambda b, k: (b, k + _KOFF, 0, 0))
    return pl.pallas_call(
        _mse_body,
        grid=(_B, _K),
        in_specs=[spec, spec, spec],
        out_specs=pl.BlockSpec((1, 2, _W), lambda b, k: (b, 0, 0)),
        out_shape=jax.ShapeDtypeStruct((_B, 2, _W), jnp.float32),
        interpret=interpret,
    )(o, m, g)


_NC, _NS = 2, 16
_NW = _NC * _NS
_WPB = _NW // _B
_CPW = _C_SC // _WPB
_QH = _H // 4
_NCH = _CPW * 4
_VECS = _W // 16

_sc_mesh = plsc.VectorSubcoreMesh(core_axis_name="c", subcore_axis_name="s")


@functools.partial(
    pl.kernel,
    out_type=jax.ShapeDtypeStruct((_NW, 2, 16), jnp.float32),
    mesh=_sc_mesh,
    cost_estimate=pl.CostEstimate(
        flops=5 * _B * _C_SC * _H * _W,
        bytes_accessed=3 * 4 * _B * _C_SC * _H * _W,
        transcendentals=0,
    ),
    scratch_types=[
        pltpu.VMEM((2, _QH, _W), jnp.float32),
        pltpu.VMEM((2, _QH, _W), jnp.float32),
        pltpu.VMEM((2, _QH, _W), jnp.float32),
        pltpu.VMEM((2, 16), jnp.float32),
        pltpu.SemaphoreType.DMA((2,)),
        pltpu.SemaphoreType.DMA((2,)),
        pltpu.SemaphoreType.DMA((2,)),
    ],
)
def _sc_partial_sums_kernel(o_hbm, m_hbm, g_hbm, out_hbm,
                            ob, mb, gb, accb, osem, msem, gsem):
    cid = lax.axis_index("c")
    sid = lax.axis_index("s")
    w = sid * _NC + cid
    b = w // _WPB
    c0 = (w % _WPB) * _CPW

    def chunk_src(hbm, q):
        c = c0 + q // 4
        h0 = (q % 4) * _QH
        return hbm.at[b, c, pl.ds(h0, _QH)]

    def start(q, slot):
        pltpu.async_copy(chunk_src(o_hbm, q), ob.at[slot], osem.at[slot])
        pltpu.async_copy(chunk_src(m_hbm, q), mb.at[slot], msem.at[slot])
        pltpu.async_copy(chunk_src(g_hbm, q), gb.at[slot], gsem.at[slot])

    def wait(q, slot):
        pltpu.make_async_copy(chunk_src(o_hbm, q), ob.at[slot], osem.at[slot]).wait()
        pltpu.make_async_copy(chunk_src(m_hbm, q), mb.at[slot], msem.at[slot]).wait()
        pltpu.make_async_copy(chunk_src(g_hbm, q), gb.at[slot], gsem.at[slot]).wait()

    start(0, 0)

    def chunk_body(q, accs):
        slot = lax.rem(q, 2)

        @pl.when(q + 1 < _NCH)
        def _():
            start(q + 1, lax.rem(q + 1, 2))

        wait(q, slot)

        def row_body(r, accs2):
            b1, b2 = accs2
            for j in range(_VECS):
                o = ob[slot, r, pl.ds(j * 16, 16)]
                m = mb[slot, r, pl.ds(j * 16, 16)]
                g = gb[slot, r, pl.ds(j * 16, 16)]
                d = (o - g) * m
                b1 = b1 + d * d
                b2 = b2 + m
            return (b1, b2)

        return lax.fori_loop(0, _QH, row_body, accs)

    z = jnp.zeros((16,), jnp.float32)
    a1, a2 = lax.fori_loop(0, _NCH, chunk_body, (z, z))
    accb[0, :] = a1
    accb[1, :] = a2
    pltpu.sync_copy(accb, out_hbm.at[w])


def kernel(output, mask, ground_truth, normalizer):
    sc_part = _sc_partial_sums_kernel(output, mask, ground_truth)
    sums = sc_part.reshape(_B, _WPB, 2, 16).sum(axis=(1, 3))
    if _C_SC < _C:
        tc_part = _tc_partial_sums(output, mask, ground_truth)
        sums = sums + tc_part.sum(axis=-1)
    loss = jnp.mean(sums[:, 0] / sums[:, 1])
    return (loss, output, ground_truth)


# speedup vs baseline: 1.1546x; 1.1546x over previous
import functools

import jax
import jax.numpy as jnp
from jax import lax
from jax.experimental import pallas as pl
from jax.experimental.pallas import tpu as pltpu
from jax.experimental.pallas import tpu_sc as plsc

_B, _C, _H, _W = 4, 96, 224, 224

_C_SC = 32

_CB = 16
_K = (_C - _C_SC) // _CB
_KOFF = _C_SC // _CB


def _mse_body(o_ref, m_ref, g_ref, out_ref):
    k = pl.program_id(1)

    @pl.when(k == 0)
    def _init():
        out_ref[...] = jnp.zeros_like(out_ref)

    o = o_ref[...]
    m = m_ref[...]
    g = g_ref[...]
    d = (o - g) * m
    s1v = jnp.sum(d * d, axis=(0, 1, 2))
    s2v = jnp.sum(m, axis=(0, 1, 2))
    out_ref[0, 0, :] += s1v
    out_ref[0, 1, :] += s2v


def _tc_partial_sums(o, m, g, interpret=False):
    spec = pl.BlockSpec((1, _CB, _H, _W), lambda b, k: (b, k + _KOFF, 0, 0))
    return pl.pallas_call(
        _mse_body,
        grid=(_B, _K),
        in_specs=[spec, spec, spec],
        out_specs=pl.BlockSpec((1, 2, _W), lambda b, k: (b, 0, 0)),
        out_shape=jax.ShapeDtypeStruct((_B, 2, _W), jnp.float32),
        interpret=interpret,
    )(o, m, g)


_NC, _NS = 2, 16
_NW = _NC * _NS
_WPB = _NW // _B
_CPW = _C_SC // _WPB
_QH = _H // 4
_NCH = _CPW * 4
_VECS = _W // 16

_sc_mesh = plsc.VectorSubcoreMesh(core_axis_name="c", subcore_axis_name="s")


@functools.partial(
    pl.kernel,
    out_type=jax.ShapeDtypeStruct((_NW, 2, 16), jnp.float32),
    mesh=_sc_mesh,
    cost_estimate=pl.CostEstimate(
        flops=5 * _B * _C_SC * _H * _W,
        bytes_accessed=3 * 4 * _B * _C_SC * _H * _W,
        transcendentals=0,
    ),
    scratch_types=[
        pltpu.VMEM((2, _QH, _W), jnp.float32),
        pltpu.VMEM((2, _QH, _W), jnp.float32),
        pltpu.VMEM((2, _QH, _W), jnp.float32),
        pltpu.VMEM((2, 16), jnp.float32),
        pltpu.SemaphoreType.DMA((2,)),
        pltpu.SemaphoreType.DMA((2,)),
        pltpu.SemaphoreType.DMA((2,)),
    ],
)
def _sc_partial_sums_kernel(o_hbm, m_hbm, g_hbm, out_hbm,
                            ob, mb, gb, accb, osem, msem, gsem):
    cid = lax.axis_index("c")
    sid = lax.axis_index("s")
    w = sid * _NC + cid
    b = w // _WPB
    c0 = (w % _WPB) * _CPW

    def chunk_src(hbm, q):
        c = c0 + q // 4
        h0 = (q % 4) * _QH
        return hbm.at[b, c, pl.ds(h0, _QH)]

    def start(q, slot):
        pltpu.async_copy(chunk_src(o_hbm, q), ob.at[slot], osem.at[slot])
        pltpu.async_copy(chunk_src(m_hbm, q), mb.at[slot], msem.at[slot])
        pltpu.async_copy(chunk_src(g_hbm, q), gb.at[slot], gsem.at[slot])

    def wait(q, slot):
        pltpu.make_async_copy(chunk_src(o_hbm, q), ob.at[slot], osem.at[slot]).wait()
        pltpu.make_async_copy(chunk_src(m_hbm, q), mb.at[slot], msem.at[slot]).wait()
        pltpu.make_async_copy(chunk_src(g_hbm, q), gb.at[slot], gsem.at[slot]).wait()

    start(0, 0)

    def chunk_body(q, accs):
        slot = lax.rem(q, 2)

        @pl.when(q + 1 < _NCH)
        def _():
            start(q + 1, lax.rem(q + 1, 2))

        wait(q, slot)

        def row_body(r, accs2):
            b1, b2 = accs2
            for j in range(_VECS):
                o = ob[slot, r, pl.ds(j * 16, 16)]
                m = mb[slot, r, pl.ds(j * 16, 16)]
                g = gb[slot, r, pl.ds(j * 16, 16)]
                d = (o - g) * m
                b1 = b1 + d * d
                b2 = b2 + m
            return (b1, b2)

        return lax.fori_loop(0, _QH, row_body, accs)

    z = jnp.zeros((16,), jnp.float32)
    a1, a2 = lax.fori_loop(0, _NCH, chunk_body, (z, z))
    accb[0, :] = a1
    accb[1, :] = a2
    pltpu.sync_copy(accb, out_hbm.at[w])


def kernel(output, mask, ground_truth, normalizer):
    sc_part = _sc_partial_sums_kernel(output, mask, ground_truth)
    sums = sc_part.reshape(_B, _WPB, 2, 16).sum(axis=(1, 3))
    if _C_SC < _C:
        tc_part = _tc_partial_sums(output, mask, ground_truth)
        sums = sums + tc_part.sum(axis=-1)
    loss = jnp.mean(sums[:, 0] / sums[:, 1])
    return (loss, output, ground_truth)
